# baseline (speedup 1.0000x reference)
import jax
import jax.numpy as jnp
from jax import lax
from jax.experimental import pallas as pl
from jax.experimental.pallas import tpu as pltpu
from jax.experimental.pallas import tpu_sc as plsc

EMB = 32
NC, NS = 2, 16
NW = NC * NS


def _gather_body(ctx_hbm, tgt_hbm, neg_hbm, win_hbm, wout_hbm,
                 ctx_out, tgt_out, neg_out,
                 slab_c, slab_t, slab_n, rows_v, sem_i, sem_g, sem_o):
    B = tgt_hbm.shape[0]
    n = B // NW
    wid = lax.axis_index("s") * NC + lax.axis_index("c")
    base = wid * n

    ci = pltpu.async_copy(ctx_hbm.at[pl.ds(6 * base, 6 * n)], slab_c, sem_i)
    ti = pltpu.async_copy(tgt_hbm.at[pl.ds(base, n)], slab_t, sem_i)
    ni = pltpu.async_copy(neg_hbm.at[pl.ds(5 * base, 5 * n)], slab_n, sem_i)
    ci.wait()
    ti.wait()
    ni.wait()

    chunks = (
        [(win_hbm, slab_c, k, ctx_out, 6 * base + k * n) for k in range(6)]
        + [(wout_hbm, slab_t, 0, tgt_out, base)]
        + [(wout_hbm, slab_n, k, neg_out, 5 * base + k * n) for k in range(5)]
    )
    out_copies = [None, None]
    for i, (tab, slab, k, out, off) in enumerate(chunks):
        buf = i % 2
        if out_copies[buf] is not None:
            out_copies[buf].wait()
        idx = slab.at[pl.ds(k * n, n)]
        pltpu.async_copy(tab.at[idx], rows_v.at[buf], sem_g).wait()
        out_copies[buf] = pltpu.async_copy(
            rows_v.at[buf], out.at[pl.ds(off, n)], sem_o)
    out_copies[0].wait()
    out_copies[1].wait()


def _dense_body(ctx_ref, tgt_ref, neg_ref, out_ref):
    @pl.when(pl.program_id(0) == 0)
    def _():
        out_ref[0, 0] = jnp.float32(0.0)
        out_ref[0, 1] = jnp.float32(0.0)

    eps = 1e-12
    cm = jnp.sum(ctx_ref[...], axis=1) * (1.0 / 6.0)
    t = tgt_ref[...]
    tt = jnp.sum(t * t, axis=1)
    cc = jnp.sum(cm * cm, axis=1)
    tc = jnp.sum(t * cm, axis=1)
    rt = lax.rsqrt(jnp.maximum(tt, eps))
    rc = lax.rsqrt(jnp.maximum(cc, eps))
    pos = jnp.sum(jax.nn.sigmoid(tc * rt * rc))
    neg = jnp.float32(0.0)
    for j in range(5):
        nrow = neg_ref[:, j, :]
        nn = jnp.sum(nrow * nrow, axis=1)
        tn = jnp.sum(t * nrow, axis=1)
        rn = lax.rsqrt(jnp.maximum(nn, eps))
        neg = neg + jnp.sum(jax.nn.sigmoid(-(tn * rt * rn)))
    out_ref[0, 0] += pos
    out_ref[0, 1] += neg


def kernel(contexts, target, negatives, W_in, W_out):
    B = contexts.shape[0]
    n = B // NW
    mesh = plsc.VectorSubcoreMesh(core_axis_name="c", subcore_axis_name="s")
    ctx_rows, tgt_rows, neg_rows = pl.kernel(
        _gather_body,
        out_type=(
            jax.ShapeDtypeStruct((6 * B, EMB), jnp.float32),
            jax.ShapeDtypeStruct((B, EMB), jnp.float32),
            jax.ShapeDtypeStruct((5 * B, EMB), jnp.float32),
        ),
        mesh=mesh,
        scratch_types=[
            pltpu.VMEM((6 * n,), jnp.int32),
            pltpu.VMEM((n,), jnp.int32),
            pltpu.VMEM((5 * n,), jnp.int32),
            pltpu.VMEM((2, n, EMB), jnp.float32),
            pltpu.SemaphoreType.DMA,
            pltpu.SemaphoreType.DMA,
            pltpu.SemaphoreType.DMA,
        ],
        compiler_params=pltpu.CompilerParams(use_tc_tiling_on_sc=False),
    )(contexts.reshape(-1).astype(jnp.int32),
      target.reshape(-1).astype(jnp.int32),
      negatives.reshape(-1).astype(jnp.int32), W_in, W_out)

    ctx3 = ctx_rows.reshape(B, 6, EMB)
    neg3 = neg_rows.reshape(B, 5, EMB)
    R = 2048
    partial = pl.pallas_call(
        _dense_body,
        grid=(B // R,),
        in_specs=[
            pl.BlockSpec((R, 6, EMB), lambda i: (i, 0, 0)),
            pl.BlockSpec((R, EMB), lambda i: (i, 0)),
            pl.BlockSpec((R, 5, EMB), lambda i: (i, 0, 0)),
        ],
        out_specs=pl.BlockSpec((1, 2), lambda i: (0, 0), memory_space=pltpu.SMEM),
        out_shape=jax.ShapeDtypeStruct((1, 2), jnp.float32),
    )(ctx3, tgt_rows, neg3)
    return partial[0, 0] / B + partial[0, 1] / (5 * B)
